# baseline (speedup 1.0000x reference)
import functools

import jax
import jax.numpy as jnp
from jax import lax
from jax.experimental import pallas as pl
from jax.experimental.pallas import tpu as pltpu
from jax.experimental.pallas import tpu_sc as plsc

N = 10000
E = 320000
H = 128
G = 64

NC = 2
NS = 16
NW = NC * NS

C = 128
K = 80
NCH = NW * K
EPAD = NCH * C
DUMMY = N
K0 = 80
K1 = 2 * K - K0
W0 = (80,)
W1 = (80,)
NP = 10112
RPT = NP // NS

_mesh = plsc.VectorSubcoreMesh(core_axis_name="c", subcore_axis_name="s",
                               num_cores=NC, num_subcores=NS)


def _deg_body(dstp, ones_hbm, zeros128, outdeg, dst_v, ones_v, acc):
    c = lax.axis_index("c")
    s = lax.axis_index("s")
    w = c * NS + s
    pltpu.sync_copy(dstp.at[pl.ds(w * K, K)], dst_v)
    pltpu.sync_copy(ones_hbm, ones_v)
    pltpu.sync_copy(zeros128, acc.at[pl.ds(s * RPT, RPT)])
    plsc.subcore_barrier()

    def body(j, carry):
        pltpu.sync_copy(ones_v, acc.at[dst_v.at[j]], add=True)
        return carry

    lax.fori_loop(0, K, body, 0)
    plsc.subcore_barrier()
    pltpu.sync_copy(acc.at[pl.ds(s * RPT, RPT)],
                    outdeg.at[c, pl.ds(s * RPT, RPT)])


_deg_kernel = pl.kernel(
    _deg_body,
    out_type=jax.ShapeDtypeStruct((NC, NP, H), jnp.float32),
    mesh=_mesh,
    scratch_types=[
        pltpu.VMEM((K, C), jnp.int32),
        pltpu.VMEM((C, H), jnp.float32),
        pltpu.VMEM_SHARED((NP, H), jnp.float32),
    ],
)


def _prop_body(y, srcp, dstp, zeros128, out, src_v, dst_v, rows_v, sem, acc):
    c = lax.axis_index("c")
    s = lax.axis_index("s")
    pltpu.sync_copy(zeros128, acc.at[pl.ds(s * RPT, RPT)])
    plsc.subcore_barrier()

    def run_windows(base, wins):
        off = 0
        for wsz in wins:
            pltpu.sync_copy(srcp.at[pl.ds(base + off, wsz)],
                            src_v.at[pl.ds(0, wsz)])
            pltpu.sync_copy(dstp.at[pl.ds(base + off, wsz)],
                            dst_v.at[pl.ds(0, wsz)])

            def body(j, carry):
                pltpu.async_copy(y.at[src_v.at[j]], rows_v, sem).wait()
                pltpu.sync_copy(rows_v, acc.at[dst_v.at[j]], add=True)
                return carry

            lax.fori_loop(0, wsz, body, 0)
            off += wsz

    @pl.when(c == 0)
    def _():
        run_windows(s * K0, W0)

    @pl.when(c == 1)
    def _():
        run_windows(NS * K0 + s * K1, W1)

    plsc.subcore_barrier()
    pltpu.sync_copy(acc.at[pl.ds(s * RPT, RPT)],
                    out.at[c, pl.ds(s * RPT, RPT)])


_prop_kernel = pl.kernel(
    _prop_body,
    out_type=jax.ShapeDtypeStruct((NC, NP, H), jnp.float32),
    mesh=_mesh,
    scratch_types=[
        pltpu.VMEM((80, C), jnp.int32),
        pltpu.VMEM((80, C), jnp.int32),
        pltpu.VMEM((C, H), jnp.float32),
        pltpu.SemaphoreType.DMA,
        pltpu.VMEM_SHARED((NP, H), jnp.float32),
    ],
)


R = 1000
NB = N // R


def _dinv_from(degp_ref):
    deg = degp_ref[0, :, 0] + degp_ref[1, :, 0] + 1.0
    return lax.rsqrt(deg)


def _b1_body(x_ref, w_ref, degp_ref, y_ref):
    dinv = _dinv_from(degp_ref)
    xw = jnp.dot(x_ref[...], w_ref[...], preferred_element_type=jnp.float32)
    y_ref[...] = xw * dinv[:, None]


def _b2_body(accp_ref, y_ref, degp_ref, b_ref, wn_ref, ynext_ref):
    dinv = _dinv_from(degp_ref)
    tot = accp_ref[0] + accp_ref[1] + y_ref[...]
    h = jnp.maximum(tot * dinv[:, None] + b_ref[...], 0.0)
    hw = jnp.dot(h, wn_ref[...], preferred_element_type=jnp.float32)
    ynext_ref[...] = hw * dinv[:, None]


def _f_body(accp_ref, y_ref, degp_ref, b_ref, batch_ref, wl1_ref, bl1_ref,
            wl2_ref, bl2_ref, out_ref, pooled_s, cnt_s):
    i = pl.program_id(0)
    dinv = _dinv_from(degp_ref)
    tot = accp_ref[0] + accp_ref[1] + y_ref[...]
    h = jnp.maximum(tot * dinv[:, None] + b_ref[...], 0.0)
    b = batch_ref[0, 0, :]
    oh = (b[:, None] == lax.broadcasted_iota(jnp.int32, (R, G), 1)
          ).astype(jnp.float32)
    p = lax.dot_general(oh, h, (((0,), (0,)), ((), ())),
                        preferred_element_type=jnp.float32)
    cnt = jnp.sum(oh, axis=0)[None, :]

    @pl.when(i == 0)
    def _():
        pooled_s[...] = p
        cnt_s[...] = cnt

    @pl.when(i > 0)
    def _():
        pooled_s[...] += p
        cnt_s[...] += cnt

    @pl.when(i == pl.num_programs(0) - 1)
    def _():
        pooled = pooled_s[...] / jnp.maximum(cnt_s[0, :], 1.0)[:, None]
        hid = jnp.maximum(
            jnp.dot(pooled, wl1_ref[...], preferred_element_type=jnp.float32)
            + bl1_ref[...], 0.0)
        out_ref[...] = (
            jnp.dot(hid, wl2_ref[...], preferred_element_type=jnp.float32)
            + bl2_ref[...])


def _rows_spec():
    return pl.BlockSpec((R, H), lambda i: (i, 0))


def _accp_spec():
    return pl.BlockSpec((2, R, H), lambda i: (0, i, 0))


def _degp_spec():
    return pl.BlockSpec((2, R, H), lambda i: (0, i, 0))


def _full(shape):
    return pl.BlockSpec(shape, lambda i: tuple(0 for _ in shape))


_b1_call = pl.pallas_call(
    _b1_body,
    grid=(NB,),
    in_specs=[_rows_spec(), _full((H, H)), _degp_spec()],
    out_specs=_rows_spec(),
    out_shape=jax.ShapeDtypeStruct((N, H), jnp.float32),
)

_b2_call = pl.pallas_call(
    _b2_body,
    grid=(NB,),
    in_specs=[_accp_spec(), _rows_spec(), _degp_spec(), _full((1, H)),
              _full((H, H))],
    out_specs=_rows_spec(),
    out_shape=jax.ShapeDtypeStruct((N, H), jnp.float32),
)

_f_call = pl.pallas_call(
    _f_body,
    grid=(NB,),
    in_specs=[_accp_spec(), _rows_spec(), _degp_spec(), _full((1, H)),
              pl.BlockSpec((1, 1, R), lambda i: (i, 0, 0)),
              _full((H, H)), _full((1, H)), _full((H, 10)), _full((1, 10))],
    out_specs=pl.BlockSpec((G, 10), lambda i: (0, 0)),
    out_shape=jax.ShapeDtypeStruct((G, 10), jnp.float32),
    scratch_shapes=[pltpu.VMEM((G, H), jnp.float32),
                    pltpu.VMEM((1, G), jnp.float32)],
)


@jax.jit
def kernel(x, edge_index, batch, W1, b1, W2, b2, W3, b3, Wl1, bl1, Wl2, bl2):
    src = edge_index[0]
    dst = edge_index[1]
    pad = EPAD - E
    srcp = jnp.concatenate(
        [src, jnp.zeros((pad,), jnp.int32)]).reshape(NCH, C)
    dstp = jnp.concatenate(
        [dst, jnp.full((pad,), DUMMY, jnp.int32)]).reshape(NCH, C)

    ones128 = jnp.ones((C, H), jnp.float32)
    zeros128 = jnp.zeros((RPT, H), jnp.float32)

    degp = _deg_kernel(dstp, ones128, zeros128)

    y = _b1_call(x, W1, degp)
    for b_l, W_next in ((b1, W2), (b2, W3)):
        accp = _prop_kernel(y, srcp, dstp, zeros128)
        y = _b2_call(accp, y, degp, b_l.reshape(1, H), W_next)
    accp = _prop_kernel(y, srcp, dstp, zeros128)

    logits = _f_call(accp, y, degp, b3.reshape(1, H),
                     batch.reshape(NB, 1, R), Wl1, bl1.reshape(1, H),
                     Wl2, bl2.reshape(1, 10))
    return (logits, jnp.zeros((), dtype=logits.dtype))


# speedup vs baseline: 1.4708x; 1.4708x over previous
import functools

import jax
import jax.numpy as jnp
from jax import lax
from jax.experimental import pallas as pl
from jax.experimental.pallas import tpu as pltpu
from jax.experimental.pallas import tpu_sc as plsc

N = 10000
E = 320000
H = 128
G = 64

NC = 2
NS = 16
NW = NC * NS

C = 128
K = 79
EPT = K * C
EPAD = NW * EPT
DUMMY = N
NP = 10112
RPT = NP // NS

_mesh = plsc.VectorSubcoreMesh(core_axis_name="c", subcore_axis_name="s",
                               num_cores=NC, num_subcores=NS)


def _deg_body(dstp, ones_hbm, zeros128, outdeg, dst_v, ones_v, acc):
    c = lax.axis_index("c")
    s = lax.axis_index("s")
    w = c * NS + s
    pltpu.sync_copy(dstp.at[w], dst_v)
    pltpu.sync_copy(ones_hbm, ones_v)
    pltpu.sync_copy(zeros128, acc.at[pl.ds(s * RPT, RPT)])
    plsc.subcore_barrier()

    def body(j, carry):
        pltpu.sync_copy(ones_v, acc.at[dst_v.at[j]], add=True)
        return carry

    lax.fori_loop(0, K, body, 0)
    plsc.subcore_barrier()
    pltpu.sync_copy(acc.at[pl.ds(s * RPT, RPT)],
                    outdeg.at[c, pl.ds(s * RPT, RPT)])


_deg_kernel = pl.kernel(
    _deg_body,
    out_type=jax.ShapeDtypeStruct((NC, NP, H), jnp.float32),
    mesh=_mesh,
    scratch_types=[
        pltpu.VMEM((K, C), jnp.int32),
        pltpu.VMEM((C, H), jnp.float32),
        pltpu.VMEM_SHARED((NP, H), jnp.float32),
    ],
)


def _prop_body(y, srcp, dstp, zeros128, out, src_v, dst_v, rows_v, sem, acc):
    c = lax.axis_index("c")
    s = lax.axis_index("s")
    w = c * NS + s
    pltpu.sync_copy(srcp.at[w], src_v)
    pltpu.sync_copy(dstp.at[w], dst_v)
    pltpu.sync_copy(zeros128, acc.at[pl.ds(s * RPT, RPT)])
    plsc.subcore_barrier()

    def body(j, carry):
        pltpu.async_copy(y.at[src_v.at[j]], rows_v, sem).wait()
        pltpu.sync_copy(rows_v, acc.at[dst_v.at[j]], add=True)
        return carry

    lax.fori_loop(0, K, body, 0)
    plsc.subcore_barrier()
    pltpu.sync_copy(acc.at[pl.ds(s * RPT, RPT)],
                    out.at[c, pl.ds(s * RPT, RPT)])


_prop_kernel = pl.kernel(
    _prop_body,
    out_type=jax.ShapeDtypeStruct((NC, NP, H), jnp.float32),
    mesh=_mesh,
    scratch_types=[
        pltpu.VMEM((K, C), jnp.int32),
        pltpu.VMEM((K, C), jnp.int32),
        pltpu.VMEM((C, H), jnp.float32),
        pltpu.SemaphoreType.DMA,
        pltpu.VMEM_SHARED((NP, H), jnp.float32),
    ],
)


R = 1000
NB = N // R


def _dinv_from(degp_ref):
    deg = degp_ref[0, :, 0] + degp_ref[1, :, 0] + 1.0
    return lax.rsqrt(deg)


def _b1_body(x_ref, w_ref, degp_ref, y_ref):
    dinv = _dinv_from(degp_ref)
    xw = jnp.dot(x_ref[...], w_ref[...], preferred_element_type=jnp.float32)
    y_ref[...] = xw * dinv[:, None]


def _b2_body(accp_ref, y_ref, degp_ref, b_ref, wn_ref, ynext_ref):
    dinv = _dinv_from(degp_ref)
    tot = accp_ref[0] + accp_ref[1] + y_ref[...]
    h = jnp.maximum(tot * dinv[:, None] + b_ref[...], 0.0)
    hw = jnp.dot(h, wn_ref[...], preferred_element_type=jnp.float32)
    ynext_ref[...] = hw * dinv[:, None]


def _f_body(accp_ref, y_ref, degp_ref, b_ref, batch_ref, wl1_ref, bl1_ref,
            wl2_ref, bl2_ref, out_ref, pooled_s, cnt_s):
    i = pl.program_id(0)
    dinv = _dinv_from(degp_ref)
    tot = accp_ref[0] + accp_ref[1] + y_ref[...]
    h = jnp.maximum(tot * dinv[:, None] + b_ref[...], 0.0)
    b = batch_ref[0, 0, :]
    oh = (b[:, None] == lax.broadcasted_iota(jnp.int32, (R, G), 1)
          ).astype(jnp.float32)
    p = lax.dot_general(oh, h, (((0,), (0,)), ((), ())),
                        preferred_element_type=jnp.float32)
    cnt = jnp.sum(oh, axis=0)[None, :]

    @pl.when(i == 0)
    def _():
        pooled_s[...] = p
        cnt_s[...] = cnt

    @pl.when(i > 0)
    def _():
        pooled_s[...] += p
        cnt_s[...] += cnt

    @pl.when(i == pl.num_programs(0) - 1)
    def _():
        pooled = pooled_s[...] / jnp.maximum(cnt_s[0, :], 1.0)[:, None]
        hid = jnp.maximum(
            jnp.dot(pooled, wl1_ref[...], preferred_element_type=jnp.float32)
            + bl1_ref[...], 0.0)
        out_ref[...] = (
            jnp.dot(hid, wl2_ref[...], preferred_element_type=jnp.float32)
            + bl2_ref[...])


def _rows_spec():
    return pl.BlockSpec((R, H), lambda i: (i, 0))


def _accp_spec():
    return pl.BlockSpec((2, R, H), lambda i: (0, i, 0))


def _degp_spec():
    return pl.BlockSpec((2, R, H), lambda i: (0, i, 0))


def _full(shape):
    return pl.BlockSpec(shape, lambda i: tuple(0 for _ in shape))


_b1_call = pl.pallas_call(
    _b1_body,
    grid=(NB,),
    in_specs=[_rows_spec(), _full((H, H)), _degp_spec()],
    out_specs=_rows_spec(),
    out_shape=jax.ShapeDtypeStruct((N, H), jnp.float32),
)

_b2_call = pl.pallas_call(
    _b2_body,
    grid=(NB,),
    in_specs=[_accp_spec(), _rows_spec(), _degp_spec(), _full((1, H)),
              _full((H, H))],
    out_specs=_rows_spec(),
    out_shape=jax.ShapeDtypeStruct((N, H), jnp.float32),
)

_f_call = pl.pallas_call(
    _f_body,
    grid=(NB,),
    in_specs=[_accp_spec(), _rows_spec(), _degp_spec(), _full((1, H)),
              pl.BlockSpec((1, 1, R), lambda i: (i, 0, 0)),
              _full((H, H)), _full((1, H)), _full((H, 10)), _full((1, 10))],
    out_specs=pl.BlockSpec((G, 10), lambda i: (0, 0)),
    out_shape=jax.ShapeDtypeStruct((G, 10), jnp.float32),
    scratch_shapes=[pltpu.VMEM((G, H), jnp.float32),
                    pltpu.VMEM((1, G), jnp.float32)],
)


@jax.jit
def kernel(x, edge_index, batch, W1, b1, W2, b2, W3, b3, Wl1, bl1, Wl2, bl2):
    src = edge_index[0]
    dst = edge_index[1]
    pad = EPAD - E
    srcp = jnp.concatenate(
        [src, jnp.zeros((pad,), jnp.int32)]).reshape(NW, K, C)
    dstp = jnp.concatenate(
        [dst, jnp.full((pad,), DUMMY, jnp.int32)]).reshape(NW, K, C)

    ones128 = jnp.ones((C, H), jnp.float32)
    zeros128 = jnp.zeros((RPT, H), jnp.float32)

    degp = _deg_kernel(dstp, ones128, zeros128)

    y = _b1_call(x, W1, degp)
    for b_l, W_next in ((b1, W2), (b2, W3)):
        accp = _prop_kernel(y, srcp, dstp, zeros128)
        y = _b2_call(accp, y, degp, b_l.reshape(1, H), W_next)
    accp = _prop_kernel(y, srcp, dstp, zeros128)

    logits = _f_call(accp, y, degp, b3.reshape(1, H),
                     batch.reshape(NB, 1, R), Wl1, bl1.reshape(1, H),
                     Wl2, bl2.reshape(1, 10))
    return (logits, jnp.zeros((), dtype=logits.dtype))
